# baseline (speedup 1.0000x reference)
import functools

import jax
import jax.numpy as jnp
from jax.experimental import pallas as pl

TEMP = 0.07
BLOCK_K = 2048


def _dist_block(x_ref, y_ref, o_ref):
    x = x_ref[...]
    y = y_ref[...]
    x_sq = jnp.sum(x * x, axis=1, keepdims=True)
    y_sq = jnp.sum(y * y, axis=1)[None, :]
    cross = jax.lax.dot_general(
        x, y, (((1,), (1,)), ((), ())),
        preferred_element_type=jnp.float32)
    d = x_sq - 2.0 * cross + y_sq
    o_ref[...] = jnp.maximum(d, 0.0) * (-1.0 / TEMP)


@jax.jit
def kernel(inputs, context):
    q, dim = inputs.shape
    k = context.shape[0]
    grid = (pl.cdiv(k, BLOCK_K),)
    return pl.pallas_call(
        _dist_block,
        grid=grid,
        in_specs=[
            pl.BlockSpec((q, dim), lambda i: (0, 0)),
            pl.BlockSpec((BLOCK_K, dim), lambda i: (i, 0)),
        ],
        out_specs=pl.BlockSpec((q, BLOCK_K), lambda i: (0, i)),
        out_shape=jax.ShapeDtypeStruct((q, k), jnp.float32),
    )(inputs, context)


# speedup vs baseline: 1.0653x; 1.0653x over previous
import functools

import jax
import jax.numpy as jnp
from jax.experimental import pallas as pl

TEMP = 0.07
BLOCK_K = 2048


def _dist_block(a_ref, b_ref, o_ref):
    cross = jax.lax.dot_general(
        a_ref[...], b_ref[...], (((1,), (0,)), ((), ())),
        preferred_element_type=jnp.float32)
    o_ref[...] = jnp.minimum(cross, 0.0)


@jax.jit
def kernel(inputs, context):
    q, dim = inputs.shape
    k = context.shape[0]
    x_sq = jnp.sum(inputs * inputs, axis=1, keepdims=True)
    y_sq = jnp.sum(context * context, axis=1)[None, :]
    a = jnp.concatenate(
        [inputs * (2.0 / TEMP), -x_sq / TEMP, jnp.ones((q, 1), jnp.float32)],
        axis=1)
    b = jnp.concatenate(
        [context.T, jnp.ones((1, k), jnp.float32), -y_sq / TEMP],
        axis=0)
    grid = (pl.cdiv(k, BLOCK_K),)
    return pl.pallas_call(
        _dist_block,
        grid=grid,
        in_specs=[
            pl.BlockSpec((q, dim + 2), lambda i: (0, 0)),
            pl.BlockSpec((dim + 2, BLOCK_K), lambda i: (0, i)),
        ],
        out_specs=pl.BlockSpec((q, BLOCK_K), lambda i: (0, i)),
        out_shape=jax.ShapeDtypeStruct((q, k), jnp.float32),
    )(a, b)


# speedup vs baseline: 1.0697x; 1.0041x over previous
import functools

import jax
import jax.numpy as jnp
from jax.experimental import pallas as pl

TEMP = 0.07
BLOCK_K = 4096


def _dist_block(a_ref, b_ref, o_ref):
    cross = jax.lax.dot_general(
        a_ref[...], b_ref[...], (((1,), (0,)), ((), ())),
        preferred_element_type=jnp.float32)
    o_ref[...] = jnp.minimum(cross, 0.0)


@jax.jit
def kernel(inputs, context):
    q, dim = inputs.shape
    k = context.shape[0]
    x_sq = jnp.sum(inputs * inputs, axis=1, keepdims=True)
    y_sq = jnp.sum(context * context, axis=1)[None, :]
    a = jnp.concatenate(
        [inputs * (2.0 / TEMP), -x_sq / TEMP, jnp.ones((q, 1), jnp.float32)],
        axis=1)
    b = jnp.concatenate(
        [context.T, jnp.ones((1, k), jnp.float32), -y_sq / TEMP],
        axis=0)
    grid = (pl.cdiv(k, BLOCK_K),)
    return pl.pallas_call(
        _dist_block,
        grid=grid,
        in_specs=[
            pl.BlockSpec((q, dim + 2), lambda i: (0, 0)),
            pl.BlockSpec((dim + 2, BLOCK_K), lambda i: (0, i)),
        ],
        out_specs=pl.BlockSpec((q, BLOCK_K), lambda i: (0, i)),
        out_shape=jax.ShapeDtypeStruct((q, k), jnp.float32),
    )(a, b)
